# baseline (speedup 1.0000x reference)
import functools

import numpy as np
import jax
import jax.numpy as jnp
from jax import lax
from jax.experimental import pallas as pl
from jax.experimental.pallas import tpu as pltpu
from jax.experimental.pallas import tpu_sc as plsc

N_NODES = 100000
N_RELS = 1000
EMB = 128
B = 16384

NC = 2
NS = 16
NW = NC * NS
BPW = B // NW
CHUNK = 64
NCHUNK = BPW // CHUNK
ROW = 2 * EMB
L = 16

_MAGIC = np.int32(0x5F3759DF)


def _rsqrt(x):
    i = plsc.bitcast(x, jnp.int32)
    y = plsc.bitcast(_MAGIC - (i >> 1), jnp.float32)
    for _ in range(2):
        y = y * (1.5 - 0.5 * x * y * y)
    return y


def _rsqrt3(x):
    i = plsc.bitcast(x, jnp.int32)
    y = plsc.bitcast(_MAGIC - (i >> 1), jnp.float32)
    w = x * y * y
    return y * (1.875 + w * (-1.25 + 0.375 * w))


def _body(s_hbm, r_hbm, o_hbm, ent_hbm, rel_hbm, out_hbm,
          sidx_v, ridx_v, oidx_v, out_v,
          bs0, bo0, br0, bs1, bo1, br1, sem0, sem1):
    cid = lax.axis_index("c")
    sid = lax.axis_index("s")
    wid = sid * NC + cid
    base = wid * BPW

    pltpu.sync_copy(s_hbm.at[pl.ds(base, BPW)], sidx_v)
    pltpu.sync_copy(r_hbm.at[pl.ds(base, BPW)], ridx_v)
    pltpu.sync_copy(o_hbm.at[pl.ds(base, BPW)], oidx_v)

    bufs = [(bs0, bo0, br0, sem0), (bs1, bo1, br1, sem1)]

    def start(c):
        bs, bo, br, sem = bufs[c % 2]
        off = c * CHUNK
        hs = pltpu.async_copy(ent_hbm.at[sidx_v.at[pl.ds(off, CHUNK)]], bs, sem)
        ho = pltpu.async_copy(ent_hbm.at[oidx_v.at[pl.ds(off, CHUNK)]], bo, sem)
        hr = pltpu.async_copy(rel_hbm.at[ridx_v.at[pl.ds(off, CHUNK)]], br, sem)
        return (hs, ho, hr)

    iota = lax.iota(jnp.int32, L)
    mask0 = iota == 0
    idx_re = [iota * 2 + 2 * L * j for j in range(EMB // L)]
    idx_im = [iota * 2 + (2 * L * j + 1) for j in range(EMB // L)]

    pending = {0: start(0)}
    for c in range(NCHUNK):
        for h in pending.pop(c):
            h.wait()
        if c + 1 < NCHUNK:
            pending[c + 1] = start(c + 1)
        bs, bo, br, _ = bufs[c % 2]
        out_off = c * CHUNK

        def one_row(rr, bs, bo, br):
            rs = jnp.full((L,), rr, jnp.int32)
            acc = jnp.zeros((L,), jnp.float32)
            for j in range(EMB // L):
                sre = plsc.load_gather(bs, [rs, idx_re[j]])
                sim = plsc.load_gather(bs, [rs, idx_im[j]])
                rre = plsc.load_gather(br, [rs, idx_re[j]])
                rim = plsc.load_gather(br, [rs, idx_im[j]])
                ore = plsc.load_gather(bo, [rs, idx_re[j]])
                oim = plsc.load_gather(bo, [rs, idx_im[j]])
                pre = sre * rre - sim * rim
                pim = sre * rim + sim * rre
                mp = pre * pre + pim * pim
                mo = ore * ore + oim * oim
                cross = pre * ore + pim * oim
                acc = acc + cross * jnp.minimum(_rsqrt3(mp * mo), 1e27)
            tot = jnp.sum(acc)
            ssq = 2.0 * EMB - (tot + tot)
            plsc.store_scatter(out_v, [rs + out_off],
                               jnp.full((L,), ssq, jnp.float32), mask=mask0)

        def row_body(rp, carry, bs=bs, bo=bo, br=br):
            for u in range(4):
                one_row(rp * 4 + u, bs, bo, br)
            return carry

        lax.fori_loop(0, CHUNK // 4, row_body, 0)

    for k in range(BPW // L):
        x = jnp.maximum(out_v[pl.ds(k * L, L)], 0.0)
        y = _rsqrt3(x)
        y = y * (1.5 - 0.5 * x * y * y)
        out_v[pl.ds(k * L, L)] = x * y

    pltpu.sync_copy(out_v, out_hbm.at[pl.ds(base, BPW)])


@jax.jit
def _run(s_idx, r_idx, o_idx, ent_weight, rel_weight):
    mesh = plsc.VectorSubcoreMesh(core_axis_name="c", subcore_axis_name="s")
    f = functools.partial(
        pl.kernel,
        mesh=mesh,
        compiler_params=pltpu.CompilerParams(use_tc_tiling_on_sc=True,
                                             needs_layout_passes=False),
        out_type=jax.ShapeDtypeStruct((B,), jnp.float32),
        scratch_types=[
            pltpu.VMEM((BPW,), jnp.int32),
            pltpu.VMEM((BPW,), jnp.int32),
            pltpu.VMEM((BPW,), jnp.int32),
            pltpu.VMEM((BPW,), jnp.float32),
            pltpu.VMEM((CHUNK, ROW), jnp.float32),
            pltpu.VMEM((CHUNK, ROW), jnp.float32),
            pltpu.VMEM((CHUNK, ROW), jnp.float32),
            pltpu.VMEM((CHUNK, ROW), jnp.float32),
            pltpu.VMEM((CHUNK, ROW), jnp.float32),
            pltpu.VMEM((CHUNK, ROW), jnp.float32),
            pltpu.SemaphoreType.DMA,
            pltpu.SemaphoreType.DMA,
        ],
    )(_body)
    return f(s_idx, r_idx, o_idx, ent_weight, rel_weight)


def kernel(s_idx, r_idx, o_idx, ent_weight, rel_weight):
    return _run(s_idx, r_idx, o_idx, ent_weight, rel_weight)


# speedup vs baseline: 1.1564x; 1.1564x over previous
import functools

import numpy as np
import jax
import jax.numpy as jnp
from jax import lax
from jax.experimental import pallas as pl
from jax.experimental.pallas import tpu as pltpu
from jax.experimental.pallas import tpu_sc as plsc

N_NODES = 100000
N_RELS = 1000
EMB = 128
B = 16384

NC = 2
NS = 16
NW = NC * NS
BPW = B // NW
CHUNK = 64
NCHUNK = BPW // CHUNK
ROW = 2 * EMB
L = 16

_MAGIC = np.int32(0x5F3759DF)


def _rsqrt(x):
    i = plsc.bitcast(x, jnp.int32)
    y = plsc.bitcast(_MAGIC - (i >> 1), jnp.float32)
    for _ in range(2):
        y = y * (1.5 - 0.5 * x * y * y)
    return y


def _rsqrt3(x):
    i = plsc.bitcast(x, jnp.int32)
    y = plsc.bitcast(_MAGIC - (i >> 1), jnp.float32)
    w = x * y * y
    return y * (1.875 + w * (-1.25 + 0.375 * w))


def _body(s_hbm, r_hbm, o_hbm, ent_hbm, rel_hbm, out_hbm,
          sidx_v, ridx_v, oidx_v, out_v, macc_v,
          bs0, bo0, br0, bs1, bo1, br1, sem0, sem1):
    cid = lax.axis_index("c")
    sid = lax.axis_index("s")
    wid = sid * NC + cid
    base = wid * BPW

    pltpu.sync_copy(s_hbm.at[pl.ds(base, BPW)], sidx_v)
    pltpu.sync_copy(r_hbm.at[pl.ds(base, BPW)], ridx_v)
    pltpu.sync_copy(o_hbm.at[pl.ds(base, BPW)], oidx_v)

    bufs = [(bs0, bo0, br0, sem0), (bs1, bo1, br1, sem1)]

    def start(c):
        bs, bo, br, sem = bufs[c % 2]
        off = c * CHUNK
        hs = pltpu.async_copy(ent_hbm.at[sidx_v.at[pl.ds(off, CHUNK)]], bs, sem)
        ho = pltpu.async_copy(ent_hbm.at[oidx_v.at[pl.ds(off, CHUNK)]], bo, sem)
        hr = pltpu.async_copy(rel_hbm.at[ridx_v.at[pl.ds(off, CHUNK)]], br, sem)
        return (hs, ho, hr)

    iota = lax.iota(jnp.int32, L)
    idx_re = [iota * 2 + 2 * L * j for j in range(EMB // L)]
    idx_im = [iota * 2 + (2 * L * j + 1) for j in range(EMB // L)]
    iota17 = iota * 17

    pending = {0: start(0)}
    for c in range(NCHUNK):
        for h in pending.pop(c):
            h.wait()
        if c + 1 < NCHUNK:
            pending[c + 1] = start(c + 1)
        bs, bo, br, _ = bufs[c % 2]
        out_off = c * CHUNK

        def one_row(rr, rb, bs, bo, br):
            rs = jnp.full((L,), rr, jnp.int32)
            acc0 = jnp.zeros((L,), jnp.float32)
            acc1 = jnp.zeros((L,), jnp.float32)
            for j in range(EMB // L):
                sre = plsc.load_gather(bs, [rs, idx_re[j]])
                sim = plsc.load_gather(bs, [rs, idx_im[j]])
                rre = plsc.load_gather(br, [rs, idx_re[j]])
                rim = plsc.load_gather(br, [rs, idx_im[j]])
                ore = plsc.load_gather(bo, [rs, idx_re[j]])
                oim = plsc.load_gather(bo, [rs, idx_im[j]])
                pre = sre * rre - sim * rim
                pim = sre * rim + sim * rre
                mp = pre * pre + pim * pim
                mo = ore * ore + oim * oim
                cross = pre * ore + pim * oim
                t = cross * jnp.minimum(_rsqrt3(mp * mo), 1e27)
                if j % 2 == 0:
                    acc0 = acc0 + t
                else:
                    acc1 = acc1 + t
            plsc.store_scatter(macc_v, [jnp.full((L,), rb * 17, jnp.int32) + iota],
                               acc0 + acc1)

        def blk_body(bk, carry, bs=bs, bo=bo, br=br):
            def row_body(rp, carry2):
                one_row(bk * L + rp * 2, rp * 2, bs, bo, br)
                one_row(bk * L + rp * 2 + 1, rp * 2 + 1, bs, bo, br)
                return carry2

            lax.fori_loop(0, L // 2, row_body, 0)
            cols = [plsc.load_gather(macc_v, [iota17 + k]) for k in range(L)]
            while len(cols) > 1:
                cols = [a + b for a, b in zip(cols[::2], cols[1::2])]
            tot = cols[0]
            ssq = 2.0 * EMB - (tot + tot)
            out_v[pl.ds(out_off + bk * L, L)] = ssq
            return carry

        lax.fori_loop(0, CHUNK // L, blk_body, 0)

    for k in range(BPW // L):
        x = jnp.maximum(out_v[pl.ds(k * L, L)], 0.0)
        y = _rsqrt3(x)
        y = y * (1.5 - 0.5 * x * y * y)
        out_v[pl.ds(k * L, L)] = x * y

    pltpu.sync_copy(out_v, out_hbm.at[pl.ds(base, BPW)])


@jax.jit
def _run(s_idx, r_idx, o_idx, ent_weight, rel_weight):
    mesh = plsc.VectorSubcoreMesh(core_axis_name="c", subcore_axis_name="s")
    f = functools.partial(
        pl.kernel,
        mesh=mesh,
        compiler_params=pltpu.CompilerParams(use_tc_tiling_on_sc=True,
                                             needs_layout_passes=False),
        out_type=jax.ShapeDtypeStruct((B,), jnp.float32),
        scratch_types=[
            pltpu.VMEM((BPW,), jnp.int32),
            pltpu.VMEM((BPW,), jnp.int32),
            pltpu.VMEM((BPW,), jnp.int32),
            pltpu.VMEM((BPW,), jnp.float32),
            pltpu.VMEM((L * 17,), jnp.float32),
            pltpu.VMEM((CHUNK, ROW), jnp.float32),
            pltpu.VMEM((CHUNK, ROW), jnp.float32),
            pltpu.VMEM((CHUNK, ROW), jnp.float32),
            pltpu.VMEM((CHUNK, ROW), jnp.float32),
            pltpu.VMEM((CHUNK, ROW), jnp.float32),
            pltpu.VMEM((CHUNK, ROW), jnp.float32),
            pltpu.SemaphoreType.DMA,
            pltpu.SemaphoreType.DMA,
        ],
    )(_body)
    return f(s_idx, r_idx, o_idx, ent_weight, rel_weight)


def kernel(s_idx, r_idx, o_idx, ent_weight, rel_weight):
    return _run(s_idx, r_idx, o_idx, ent_weight, rel_weight)


# speedup vs baseline: 1.1564x; 1.0000x over previous
import functools

import numpy as np
import jax
import jax.numpy as jnp
from jax import lax
from jax.experimental import pallas as pl
from jax.experimental.pallas import tpu as pltpu
from jax.experimental.pallas import tpu_sc as plsc

N_NODES = 100000
N_RELS = 1000
EMB = 128
B = 16384

NC = 2
NS = 16
NW = NC * NS
BPW = B // NW
CHUNK = 64
NCHUNK = BPW // CHUNK
ROW = 2 * EMB
L = 16

_MAGIC = np.int32(0x5F3759DF)


def _rsqrt(x):
    i = plsc.bitcast(x, jnp.int32)
    y = plsc.bitcast(_MAGIC - (i >> 1), jnp.float32)
    for _ in range(2):
        y = y * (1.5 - 0.5 * x * y * y)
    return y


def _rsqrt3(x):
    i = plsc.bitcast(x, jnp.int32)
    y = plsc.bitcast(_MAGIC - (i >> 1), jnp.float32)
    w = x * y * y
    return y * (1.875 + w * (-1.25 + 0.375 * w))


def _body(s_hbm, r_hbm, o_hbm, ent_hbm, rel_hbm, out_hbm,
          sidx_v, ridx_v, oidx_v, out_v, macc_v,
          bs0, bo0, br0, bs1, bo1, br1, sem0, sem1):
    cid = lax.axis_index("c")
    sid = lax.axis_index("s")
    wid = sid * NC + cid
    base = wid * BPW

    pltpu.sync_copy(s_hbm.at[pl.ds(base, BPW)], sidx_v)
    pltpu.sync_copy(r_hbm.at[pl.ds(base, BPW)], ridx_v)
    pltpu.sync_copy(o_hbm.at[pl.ds(base, BPW)], oidx_v)

    bufs = [(bs0, bo0, br0, sem0), (bs1, bo1, br1, sem1)]

    def start(c):
        bs, bo, br, sem = bufs[c % 2]
        off = c * CHUNK
        hs = pltpu.async_copy(ent_hbm.at[sidx_v.at[pl.ds(off, CHUNK)]], bs, sem)
        ho = pltpu.async_copy(ent_hbm.at[oidx_v.at[pl.ds(off, CHUNK)]], bo, sem)
        hr = pltpu.async_copy(rel_hbm.at[ridx_v.at[pl.ds(off, CHUNK)]], br, sem)
        return (hs, ho, hr)

    iota = lax.iota(jnp.int32, L)
    idx_re = [iota * 2 + 2 * L * j for j in range(EMB // L)]
    idx_im = [iota * 2 + (2 * L * j + 1) for j in range(EMB // L)]
    iota17 = iota * 17

    pending = {0: start(0)}
    for c in range(NCHUNK):
        for h in pending.pop(c):
            h.wait()
        if c + 1 < NCHUNK:
            pending[c + 1] = start(c + 1)
        bs, bo, br, _ = bufs[c % 2]
        out_off = c * CHUNK

        def one_row(rr, rb, bs, bo, br):
            rs = jnp.full((L,), rr, jnp.int32)
            acc0 = jnp.zeros((L,), jnp.float32)
            acc1 = jnp.zeros((L,), jnp.float32)
            for j in range(EMB // L):
                sre = plsc.load_gather(bs, [rs, idx_re[j]])
                sim = plsc.load_gather(bs, [rs, idx_im[j]])
                rre = plsc.load_gather(br, [rs, idx_re[j]])
                rim = plsc.load_gather(br, [rs, idx_im[j]])
                ore = plsc.load_gather(bo, [rs, idx_re[j]])
                oim = plsc.load_gather(bo, [rs, idx_im[j]])
                pre = sre * rre - sim * rim
                pim = sre * rim + sim * rre
                mp = pre * pre + pim * pim
                mo = ore * ore + oim * oim
                cross = pre * ore + pim * oim
                t = cross * jnp.minimum(_rsqrt3(mp * mo), 1e27)
                if j % 2 == 0:
                    acc0 = acc0 + t
                else:
                    acc1 = acc1 + t
            plsc.store_scatter(macc_v, [jnp.full((L,), rb * 17, jnp.int32) + iota],
                               acc0 + acc1)

        def blk_body(bk, carry, bs=bs, bo=bo, br=br):
            def row_body(rp, carry2):
                one_row(bk * L + rp * 2, rp * 2, bs, bo, br)
                one_row(bk * L + rp * 2 + 1, rp * 2 + 1, bs, bo, br)
                return carry2

            lax.fori_loop(0, L // 2, row_body, 0)
            cols = [plsc.load_gather(macc_v, [iota17 + k]) for k in range(L)]
            while len(cols) > 1:
                cols = [a + b for a, b in zip(cols[::2], cols[1::2])]
            tot = cols[0]
            ssq = 2.0 * EMB - (tot + tot)
            out_v[pl.ds(out_off + bk * L, L)] = ssq
            return carry

        lax.fori_loop(0, CHUNK // L, blk_body, 0)

    for k in range(BPW // L):
        x = jnp.maximum(out_v[pl.ds(k * L, L)], 0.0)
        y = _rsqrt3(x)
        y = y * (1.5 - 0.5 * x * y * y)
        out_v[pl.ds(k * L, L)] = x * y

    pltpu.sync_copy(out_v, out_hbm.at[pl.ds(base, BPW)])


@jax.jit
def _run(s_idx, r_idx, o_idx, ent_weight, rel_weight):
    mesh = plsc.VectorSubcoreMesh(core_axis_name="c", subcore_axis_name="s")
    f = functools.partial(
        pl.kernel,
        mesh=mesh,
        compiler_params=pltpu.CompilerParams(use_tc_tiling_on_sc=True,
                                             needs_layout_passes=False,
                                             disable_bounds_checks=True,
                                             skip_device_barrier=True),
        out_type=jax.ShapeDtypeStruct((B,), jnp.float32),
        scratch_types=[
            pltpu.VMEM((BPW,), jnp.int32),
            pltpu.VMEM((BPW,), jnp.int32),
            pltpu.VMEM((BPW,), jnp.int32),
            pltpu.VMEM((BPW,), jnp.float32),
            pltpu.VMEM((L * 17,), jnp.float32),
            pltpu.VMEM((CHUNK, ROW), jnp.float32),
            pltpu.VMEM((CHUNK, ROW), jnp.float32),
            pltpu.VMEM((CHUNK, ROW), jnp.float32),
            pltpu.VMEM((CHUNK, ROW), jnp.float32),
            pltpu.VMEM((CHUNK, ROW), jnp.float32),
            pltpu.VMEM((CHUNK, ROW), jnp.float32),
            pltpu.SemaphoreType.DMA,
            pltpu.SemaphoreType.DMA,
        ],
    )(_body)
    return f(s_idx, r_idx, o_idx, ent_weight, rel_weight)


def kernel(s_idx, r_idx, o_idx, ent_weight, rel_weight):
    return _run(s_idx, r_idx, o_idx, ent_weight, rel_weight)
